# Initial kernel scaffold; baseline (speedup 1.0000x reference)
import jax
import jax.numpy as jnp
from jax.experimental import pallas as pl


def kernel(x_s, edge_index_s, edge_weight_s, edge_index_s1, edge_weight_s1, x_t, conv0_W, conv0_b, bnc0_g, bnc0_b, conv1_W, conv1_b, bnc1_g, bnc1_b, lin0_W, lin0_b, lin1_W, lin1_b, bnm1_g, bnm1_b, lin2_W, lin2_b, bnm2_g, bnm2_b, lin3_W, lin3_b):
    raise NotImplementedError("write your pallas kernel here")


# speedup vs baseline: 1.5754x; 1.5754x over previous
import functools

import jax
import jax.numpy as jnp
from jax.experimental import pallas as pl
from jax.experimental.pallas import tpu as pltpu

B = 32
EDGE_NUM = 8978
K = 4
N0 = B * EDGE_NUM
N1 = N0 // 2
SLOPE = 0.33
EPS = 1e-5
INV = 1.0 / (1.0 + EPS) ** 0.5

RB_A = 8192
N1P = ((N1 + RB_A - 1) // RB_A) * RB_A
RB_C = 8192
N1PC = N1P


def _lrelu(x):
    return jnp.where(x >= 0, x, SLOPE * x)


def _combineA_body(tpair_ref, wa_ref, wb_ref, bias_ref, out_ref):
    tp = tpair_ref[...]
    ze = jnp.dot(tp, wa_ref[...], preferred_element_type=jnp.float32, precision=jax.lax.Precision.HIGHEST)
    zo = jnp.dot(tp, wb_ref[...], preferred_element_type=jnp.float32, precision=jax.lax.Precision.HIGHEST)
    z = jnp.maximum(ze, zo) + bias_ref[...]
    out_ref[...] = _lrelu(z)


def _combineA(tpair, wa, wb, bias):
    nb = N1P // RB_A
    return pl.pallas_call(
        _combineA_body,
        grid=(nb,),
        in_specs=[
            pl.BlockSpec((RB_A, 8), lambda i: (i, 0)),
            pl.BlockSpec((8, 32), lambda i: (0, 0)),
            pl.BlockSpec((8, 32), lambda i: (0, 0)),
            pl.BlockSpec((1, 32), lambda i: (0, 0)),
        ],
        out_specs=pl.BlockSpec((RB_A, 32), lambda i: (i, 0)),
        out_shape=jax.ShapeDtypeStruct((N1P, 32), jnp.float32),
    )(tpair, wa, wb, bias)


def _combineC_body(tcat_ref, wc_ref, bias_ref, l0w_ref, l0b_ref, out_ref):
    z = jnp.dot(tcat_ref[...], wc_ref[...], preferred_element_type=jnp.float32, precision=jax.lax.Precision.HIGHEST)
    z = _lrelu(z + bias_ref[...])
    r = jnp.dot(z, l0w_ref[...], preferred_element_type=jnp.float32, precision=jax.lax.Precision.HIGHEST)
    out_ref[...] = jax.nn.relu(r + l0b_ref[0, 0])


def _combineC(tcat, wc, bias, l0w, l0b):
    nb = N1PC // RB_C
    return pl.pallas_call(
        _combineC_body,
        grid=(nb,),
        in_specs=[
            pl.BlockSpec((RB_C, 4 * 32), lambda i: (i, 0)),
            pl.BlockSpec((4 * 32, 32), lambda i: (0, 0)),
            pl.BlockSpec((1, 32), lambda i: (0, 0)),
            pl.BlockSpec((32, 1), lambda i: (0, 0)),
            pl.BlockSpec((1, 1), lambda i: (0, 0), memory_space=pltpu.SMEM),
        ],
        out_specs=pl.BlockSpec((RB_C, 1), lambda i: (i, 0)),
        out_shape=jax.ShapeDtypeStruct((N1PC, 1), jnp.float32),
    )(tcat, wc, bias, l0w, l0b)


def _mlp_body(r_ref, w1_ref, b1_ref, s1_ref, t1_ref,
              w2_ref, b2_ref, s2_ref, t2_ref, w3_ref, b3_ref, out_ref):
    z = jnp.dot(r_ref[...], w1_ref[...], preferred_element_type=jnp.float32, precision=jax.lax.Precision.HIGHEST)
    z = jax.nn.relu((z + b1_ref[...]) * s1_ref[...] + t1_ref[...])
    z = jnp.dot(z, w2_ref[...], preferred_element_type=jnp.float32, precision=jax.lax.Precision.HIGHEST)
    z = jax.nn.relu((z + b2_ref[...]) * s2_ref[...] + t2_ref[...])
    z = jnp.dot(z, w3_ref[...], preferred_element_type=jnp.float32, precision=jax.lax.Precision.HIGHEST)
    out_ref[...] = z + b3_ref[0, 0]


def _mlp(r, w1, b1, s1, t1, w2, b2, s2, t2, w3, b3):
    return pl.pallas_call(
        _mlp_body,
        in_specs=[
            pl.BlockSpec(memory_space=pltpu.VMEM),
            pl.BlockSpec(memory_space=pltpu.VMEM),
            pl.BlockSpec(memory_space=pltpu.VMEM),
            pl.BlockSpec(memory_space=pltpu.VMEM),
            pl.BlockSpec(memory_space=pltpu.VMEM),
            pl.BlockSpec(memory_space=pltpu.VMEM),
            pl.BlockSpec(memory_space=pltpu.VMEM),
            pl.BlockSpec(memory_space=pltpu.VMEM),
            pl.BlockSpec(memory_space=pltpu.VMEM),
            pl.BlockSpec(memory_space=pltpu.VMEM),
            pl.BlockSpec((1, 1), memory_space=pltpu.SMEM),
        ],
        out_specs=pl.BlockSpec(memory_space=pltpu.VMEM),
        out_shape=jax.ShapeDtypeStruct((B, 1), jnp.float32),
    )(r, w1, b1, s1, t1, w2, b2, s2, t2, w3, b3)


def _spmm(x, ei, ew):
    src, dst = ei[0], ei[1]
    msgs = ew[:, None] * jnp.take(x, src, axis=0) if x.ndim == 2 else ew * jnp.take(x, src)
    return jax.ops.segment_sum(msgs, dst, num_segments=x.shape[0])


def kernel(x_s, edge_index_s, edge_weight_s, edge_index_s1, edge_weight_s1, x_t,
           conv0_W, conv0_b, bnc0_g, bnc0_b,
           conv1_W, conv1_b, bnc1_g, bnc1_b,
           lin0_W, lin0_b, lin1_W, lin1_b, bnm1_g, bnm1_b,
           lin2_W, lin2_b, bnm2_g, bnm2_b, lin3_W, lin3_b):
    t0 = x_s[:, 0]
    t1 = t0 - _spmm(t0, edge_index_s, edge_weight_s)
    t2 = (3.0 * t1 - _spmm(t1, edge_index_s, edge_weight_s) - t0) * 0.5
    t3 = (5.0 * t2 - _spmm(t2, edge_index_s, edge_weight_s) - 2.0 * t1) / 3.0

    s0 = INV * bnc0_g
    w0 = conv0_W[:, 0, :] * s0[None, :]
    b0 = (conv0_b * s0 + bnc0_b)[None, :]
    tpair = jnp.stack([t0, t1, t2, t3], axis=-1).reshape(N1, 8)
    tpair = jnp.pad(tpair, ((0, N1P - N1), (0, 0)))
    z4 = jnp.zeros((4, 32), jnp.float32)
    wa = jnp.concatenate([w0, z4], axis=0)
    wb = jnp.concatenate([z4, w0], axis=0)
    x1 = _combineA(tpair, wa, wb, b0)[:N1]

    u0 = x1
    u1 = u0 - _spmm(u0, edge_index_s1, edge_weight_s1)
    u2 = (3.0 * u1 - _spmm(u1, edge_index_s1, edge_weight_s1) - u0) * 0.5
    u3 = (5.0 * u2 - _spmm(u2, edge_index_s1, edge_weight_s1) - 2.0 * u1) / 3.0

    s1c = INV * bnc1_g
    wc = jnp.concatenate([conv1_W[k] * s1c[None, :] for k in range(K)], axis=0)
    bc = (conv1_b * s1c + bnc1_b)[None, :]
    tcat = jnp.concatenate([u0, u1, u2, u3], axis=1)
    tcat = jnp.pad(tcat, ((0, N1PC - N1), (0, 0)))
    r = _combineC(tcat, wc, bc, lin0_W, lin0_b.reshape(1, 1))[:N1]

    rmat = r.reshape(B, EDGE_NUM // 2)
    out = _mlp(rmat, lin1_W, lin1_b[None, :], (INV * bnm1_g)[None, :], bnm1_b[None, :],
               lin2_W, lin2_b[None, :], (INV * bnm2_g)[None, :], bnm2_b[None, :],
               lin3_W, lin3_b.reshape(1, 1))
    return out
